# baseline (speedup 1.0000x reference)
import jax
import jax.numpy as jnp
from jax import lax
from jax.experimental import pallas as pl
from jax.experimental.pallas import tpu as pltpu
from jax.experimental.pallas import tpu_sc as plsc

N_CB = 256
D = 10000
B = 4096
L = 16
NC = 2
NS = 16
NW = NC * NS
BPW = B // NW
DPAD = 10112


def _body(mem_hbm, item_hbm, pos_hbm, gate_hbm, codes_hbm, out_hbm,
          idx_v, gate_v, code_v, mem_v, item_v, out_v, in_sem, g_sem):
    wid = lax.axis_index("s") * NC + lax.axis_index("c")
    base = wid * BPW

    pltpu.sync_copy(pos_hbm.at[pl.ds(base * 8, BPW * 8)], idx_v)
    pltpu.sync_copy(gate_hbm.at[pl.ds(base * L, BPW * L)], gate_v)

    ncb = jnp.full((L,), N_CB, jnp.int32)
    for i in range(BPW * 8 // L):
        s = pl.ds(i * L, L)
        idx_v[s] = lax.rem(idx_v[s], ncb)

    def row(r, carry):
        cp_c = pltpu.async_copy(
            codes_hbm.at[idx_v.at[pl.ds(pl.multiple_of(r * 8, 8), 1)]], code_v, g_sem)
        cp_m = pltpu.async_copy(mem_hbm.at[pl.ds(base + r, 1)], mem_v, in_sem)
        cp_i = pltpu.async_copy(item_hbm.at[pl.ds(base + r, 1)], item_v, in_sem)

        g = gate_v[pl.ds(pl.multiple_of(r * L, L), L)]
        one_m_g = 1.0 - g

        cp_c.wait()
        cp_m.wait()
        cp_i.wait()

        def col(j, c2):
            cs = pl.ds(j * L, L)
            out_v[0, cs] = g * mem_v[0, cs] + one_m_g * (item_v[0, cs] * code_v[0, cs])
            return c2

        lax.fori_loop(0, D // L, col, 0)
        pltpu.sync_copy(out_v, out_hbm.at[pl.ds(base + r, 1)])
        return carry

    lax.fori_loop(0, BPW, row, 0)


@jax.jit
def kernel(memory, item, position, gate, position_codes):
    pos8 = jnp.repeat(position.astype(jnp.int32), 8)
    gate16 = jnp.repeat(gate.reshape(-1), L)
    codes_pad = jnp.pad(position_codes, ((0, 0), (0, DPAD - D)))
    f = pl.kernel(
        _body,
        out_type=jax.ShapeDtypeStruct((B, D), jnp.float32),
        mesh=plsc.VectorSubcoreMesh(core_axis_name="c", subcore_axis_name="s"),
        scratch_types=[
            pltpu.VMEM((BPW * 8,), jnp.int32),
            pltpu.VMEM((BPW * L,), jnp.float32),
            pltpu.VMEM((1, DPAD), jnp.float32),
            pltpu.VMEM((1, D), jnp.float32),
            pltpu.VMEM((1, D), jnp.float32),
            pltpu.VMEM((1, D), jnp.float32),
            pltpu.SemaphoreType.DMA,
            pltpu.SemaphoreType.DMA,
        ],
    )
    return f(memory, item, pos8, gate16, codes_pad)
